# baseline (speedup 1.0000x reference)
import functools
import jax
import jax.numpy as jnp
from jax import lax
from jax.experimental import pallas as pl
from jax.experimental.pallas import tpu as pltpu
from jax.experimental.pallas import tpu_sc as plsc

N = 4096
E = 131072
NE = E + N
C = 2
NC, NS, L = 2, 16, 16
EPT = NE // NS
CHUNK = 128
NCHUNK = EPT // CHUNK
NBUF = 6
ROWS_PT = N // (NC * NS)
SL = N // NS
W = 80

_MESH = plsc.VectorSubcoreMesh(core_axis_name="c", subcore_axis_name="s",
                               num_cores=NC, num_subcores=NS)

_SC_PARAMS = pltpu.CompilerParams(needs_layout_passes=False)
_SC_PARAMS_LINEAR = pltpu.CompilerParams(needs_layout_passes=False,
                                         use_tc_tiling_on_sc=False)

_DN = jax.lax.GatherDimensionNumbers(
    offset_dims=(), collapsed_slice_dims=(0,), start_index_map=(0,))


def _vgather16(vec, idx):
    return jax.lax.gather(
        vec, idx[:, None], _DN, slice_sizes=(1,),
        mode=jax.lax.GatherScatterMode.PROMISE_IN_BOUNDS)


def _sc_build_p(src_hbm, dst_hbm, ones_hbm, zf_hbm, p_hbm,
                svb, dvb, iv, onesv):
    cid = lax.axis_index("c")
    sid = lax.axis_index("s")
    base_w = (cid * NS + sid) * ROWS_PT * N

    def zero_body(j, _):
        pltpu.sync_copy(zf_hbm, p_hbm.at[pl.ds(base_w + j * 16384, 16384)])
        return 0
    lax.fori_loop(0, (ROWS_PT * N) // 16384, zero_body, 0)
    plsc.subcore_barrier()

    e0 = sid * EPT
    pltpu.sync_copy(src_hbm.at[pl.ds(e0, EPT)], svb)
    pltpu.sync_copy(dst_hbm.at[pl.ds(e0, EPT)], dvb)
    pltpu.sync_copy(ones_hbm, onesv)

    def chunk_body(i, _):
        for k in range(CHUNK // L):
            sv = svb[pl.ds(i * CHUNK + k * L, L)]
            dv = dvb[pl.ds(i * CHUNK + k * L, L)]
            iv[pl.ds(k * L, L)] = sv * N + dv
        pltpu.sync_copy(onesv, p_hbm.at[iv])
        return 0
    lax.fori_loop(0, NCHUNK, chunk_body, 0)


@functools.partial(
    pl.kernel,
    out_type=jax.ShapeDtypeStruct((N * N,), jnp.float32),
    mesh=_MESH,
    scratch_types=[
        pltpu.VMEM((EPT,), jnp.int32),
        pltpu.VMEM((EPT,), jnp.int32),
        pltpu.VMEM((CHUNK,), jnp.int32),
        pltpu.VMEM((CHUNK,), jnp.float32),
    ],
)
def _build_p_kernel(src_hbm, dst_hbm, ones_hbm, zf_hbm, p_hbm,
                    svb, dvb, iv, onesv):
    _sc_build_p(src_hbm, dst_hbm, ones_hbm, zf_hbm, p_hbm,
                svb, dvb, iv, onesv)


def _sc_value(src_hbm, dst_hbm, typ_hbm, lut_hbm, z2_hbm, zw_hbm,
              u1_hbm, u2_hbm, u3_hbm,
              svb, dvb, tvb, lutc, slicev, bufs, acc_sp):
    cid = lax.axis_index("c")
    sid = lax.axis_index("s")
    e0 = sid * EPT
    pltpu.sync_copy(src_hbm.at[pl.ds(e0, EPT)], svb)
    pltpu.sync_copy(dst_hbm.at[pl.ds(e0, EPT)], dvb)
    pltpu.sync_copy(typ_hbm.at[pl.ds(e0, EPT)], tvb)
    pltpu.sync_copy(lut_hbm.at[:, cid], lutc)

    pltpu.sync_copy(zw_hbm, acc_sp.at[pl.ds(sid * SL, SL)])
    plsc.subcore_barrier()

    row_off = cid * N
    iota = jax.lax.iota(jnp.int32, L)
    NV = W // L

    def _prep(i, bix, lp):
        idxg, scidx, wbuf, _, _ = bufs[bix]
        for k in range(CHUNK // L):
            sv = svb[pl.ds(i * CHUNK + k * L, L)]
            dv = dvb[pl.ds(i * CHUNK + k * L, L)]
            tv = tvb[pl.ds(i * CHUNK + k * L, L)]
            idxg[pl.ds(k * L, L)] = sv + row_off
            scidx[pl.ds(k * L, L)] = dv
            wbuf[pl.ds(k * L, L)] = _vgather16(lp, tv)

    def _start(bix, tbl):
        idxg, _, _, rows, sem = bufs[bix]
        pltpu.async_copy(tbl.at[idxg], rows, sem)

    def _finish(bix, tbl):
        idxg, scidx, wbuf, rows, sem = bufs[bix]
        pltpu.make_async_copy(tbl.at[idxg], rows, sem).wait()

        def q_body(q, _):
            for u in range(4):
                e16 = jnp.full((L,), q * 4 + u, jnp.int32)
                bw = plsc.load_gather(wbuf, [e16])
                for v in range(NV):
                    c16 = iota + (v * L)
                    x = plsc.load_gather(rows, [e16, c16])
                    plsc.store_scatter(rows, [e16, c16], x * bw)
            return 0
        lax.fori_loop(0, CHUNK // 4, q_body, 0)
        pltpu.sync_copy(rows, acc_sp.at[scidx], add=True)

    ones_lane0 = jnp.where(iota == 0, 1.0, 0.0).astype(jnp.float32)

    for p, (tbl, out) in enumerate([(z2_hbm, u1_hbm), (u1_hbm, u2_hbm),
                                    (u2_hbm, u3_hbm)]):
        lp = lutc[p]
        for j in range(NBUF):
            _prep(j, j, lp)
            _start(j, tbl)

        def ring_body(g, _, lp=lp, tbl=tbl):
            for j in range(NBUF):
                _finish(j, tbl)

                @pl.when(g < NCHUNK // NBUF - 1)
                def _(j=j, g=g):
                    _prep((g + 1) * NBUF + j, j, lp)
                    _start(j, tbl)
            return 0
        lax.fori_loop(0, NCHUNK // NBUF, ring_body, 0)
        plsc.subcore_barrier()

        if p == 1:
            pltpu.sync_copy(acc_sp.at[pl.ds(sid * SL, SL)], slicev)

            def fin_body(r, _):
                r16 = jnp.full((L,), r, jnp.int32)
                s0r = plsc.load_gather(slicev, [r16, jnp.full((L,), 64,
                                                              jnp.int32)])
                inv = jnp.where(s0r == 0, 0.0,
                                1.0 / jnp.where(s0r == 0, 1.0, s0r))
                for v in range(4):
                    c16 = iota + (v * L)
                    x = plsc.load_gather(slicev, [r16, c16])
                    plsc.store_scatter(slicev, [r16, c16], x * inv)
                plsc.store_scatter(slicev, [r16, iota + 4 * L], ones_lane0)
                return 0
            lax.fori_loop(0, SL, fin_body, 0)
            pltpu.sync_copy(slicev,
                            out.at[pl.ds(row_off + sid * SL, SL)])
        else:
            pltpu.sync_copy(acc_sp.at[pl.ds(sid * SL, SL)],
                            out.at[pl.ds(row_off + sid * SL, SL)])
        pltpu.sync_copy(zw_hbm, acc_sp.at[pl.ds(sid * SL, SL)])
        plsc.subcore_barrier()


def _value_scratch():
    s = [
        pltpu.VMEM((EPT,), jnp.int32),
        pltpu.VMEM((EPT,), jnp.int32),
        pltpu.VMEM((EPT,), jnp.int32),
        pltpu.VMEM((3, L), jnp.float32),
        pltpu.VMEM((SL, W), jnp.float32),
    ]
    for _ in range(NBUF):
        s += [pltpu.VMEM((CHUNK,), jnp.int32),
              pltpu.VMEM((CHUNK,), jnp.int32),
              pltpu.VMEM((CHUNK,), jnp.float32),
              pltpu.VMEM((CHUNK, W), jnp.float32),
              pltpu.SemaphoreType.DMA]
    s.append(pltpu.VMEM_SHARED((N, W), jnp.float32))
    return s


@functools.partial(
    pl.kernel,
    out_type=(jax.ShapeDtypeStruct((C * N, W), jnp.float32),
              jax.ShapeDtypeStruct((C * N, W), jnp.float32),
              jax.ShapeDtypeStruct((C * N, W), jnp.float32)),
    mesh=_MESH,
    compiler_params=_SC_PARAMS_LINEAR,
    scratch_types=_value_scratch(),
)
def _value_kernel(src_hbm, dst_hbm, typ_hbm, lut_hbm, z2_hbm, zw_hbm,
                  u1_hbm, u2_hbm, u3_hbm, svb, dvb, tvb, lutc, slicev,
                  *rest):
    bufs = tuple(tuple(rest[5 * i:5 * i + 5]) for i in range(NBUF))
    acc_sp = rest[5 * NBUF]
    _sc_value(src_hbm, dst_hbm, typ_hbm, lut_hbm, z2_hbm, zw_hbm,
              u1_hbm, u2_hbm, u3_hbm,
              svb, dvb, tvb, lutc, slicev, bufs, acc_sp)


def _prep_body(w1_ref, w2_ref, w3_ref, h_ref, wg_ref, lut_ref, xh_ref):
    def lutify(w):
        f = jax.nn.softmax(w, axis=0)
        return jnp.concatenate([f, jnp.zeros((L - 5, C), f.dtype)], axis=0).T
    lut_ref[0] = lutify(w1_ref[...])
    lut_ref[1] = lutify(w2_ref[...])
    lut_ref[2] = lutify(w3_ref[...])
    xh_ref[...] = jax.lax.dot_general(
        h_ref[...], wg_ref[...], (((1,), (0,)), ((), ())),
        precision=jax.lax.Precision.HIGHEST,
        preferred_element_type=jnp.float32)


def _tc_prep(w1, w2, w3, h, wg):
    return pl.pallas_call(
        _prep_body,
        out_shape=(jax.ShapeDtypeStruct((3, C, L), jnp.float32),
                   jax.ShapeDtypeStruct((N, 64), jnp.float32)),
    )(w1, w2, w3, h, wg)


def _conv_body(p_ref, o_ref):
    o_ref[...] = p_ref[...].astype(jnp.bfloat16)


def _tc_conv(p2d):
    blk = 512
    return pl.pallas_call(
        _conv_body,
        grid=(N // blk,),
        in_specs=[pl.BlockSpec((blk, N), lambda i: (i, 0))],
        out_specs=pl.BlockSpec((blk, N), lambda i: (i, 0)),
        out_shape=jax.ShapeDtypeStruct((N, N), jnp.bfloat16),
    )(p2d)


_BM = 1024


def _c2_body(a_ref, b_ref, o_ref):
    acc = jax.lax.dot_general(
        a_ref[...], b_ref[...], (((1,), (0,)), ((), ())),
        preferred_element_type=jnp.float32)
    o_ref[...] = (acc > 0).astype(jnp.bfloat16)


def _tc_c2(pbf):
    return pl.pallas_call(
        _c2_body,
        grid=(N // _BM, N // _BM),
        in_specs=[pl.BlockSpec((_BM, N), lambda i, j: (i, 0)),
                  pl.BlockSpec((N, _BM), lambda i, j: (0, j))],
        out_specs=pl.BlockSpec((_BM, _BM), lambda i, j: (i, j)),
        out_shape=jax.ShapeDtypeStruct((N, N), jnp.bfloat16),
        compiler_params=pltpu.CompilerParams(
            dimension_semantics=("arbitrary", "arbitrary")),
    )(pbf, pbf)


def _c3_body(a_ref, b_ref, dro_ref, dci_ref, col_acc):
    i = pl.program_id(0)
    j = pl.program_id(1)
    acc = jax.lax.dot_general(
        a_ref[...], b_ref[...], (((1,), (0,)), ((), ())),
        preferred_element_type=jnp.float32)
    ind = (acc > 0).astype(jnp.float32)

    @pl.when(j == 0)
    def _():
        dro_ref[...] = jnp.zeros_like(dro_ref)
    dro_ref[...] += jnp.sum(ind, axis=1)

    @pl.when(jnp.logical_and(i == 0, j == 0))
    def _():
        col_acc[...] = jnp.zeros_like(col_acc)
    col_acc[pl.ds(j * _BM, _BM)] += jnp.sum(ind, axis=0)

    @pl.when(jnp.logical_and(i == pl.num_programs(0) - 1,
                             j == pl.num_programs(1) - 1))
    def _():
        dci_ref[...] = col_acc[...]


def _tc_c3(p2bf, pbf):
    return pl.pallas_call(
        _c3_body,
        grid=(N // _BM, N // _BM),
        in_specs=[pl.BlockSpec((_BM, N), lambda i, j: (i, 0)),
                  pl.BlockSpec((N, _BM), lambda i, j: (0, j))],
        out_specs=(pl.BlockSpec((_BM,), lambda i, j: (i,)),
                   pl.BlockSpec((N,), lambda i, j: (0,))),
        out_shape=(jax.ShapeDtypeStruct((N,), jnp.float32),
                   jax.ShapeDtypeStruct((N,), jnp.float32)),
        scratch_shapes=[pltpu.VMEM((N,), jnp.float32)],
        compiler_params=pltpu.CompilerParams(
            dimension_semantics=("arbitrary", "arbitrary")),
    )(p2bf, pbf)


def _mid_body(dro_ref, dci_ref, xh_ref, z2_ref, nd_ref):
    dro = dro_ref[...]
    dci = dci_ref[...]
    ns = jnp.where(dro > 0, jax.lax.rsqrt(jnp.maximum(dro, 1.0)), 0.0)
    nd_ref[...] = jnp.where(dci > 0, jax.lax.rsqrt(jnp.maximum(dci, 1.0)),
                            0.0)
    z = ns[:, None] * xh_ref[...]
    col = jnp.concatenate(
        [jnp.ones((N, 1), jnp.float32), jnp.zeros((N, W - 65), jnp.float32)],
        axis=1)
    zrow = jnp.concatenate([z, col], axis=1)
    z2_ref[pl.ds(0, N), :] = zrow
    z2_ref[pl.ds(N, N), :] = zrow


def _tc_mid(dro, dci, xh):
    return pl.pallas_call(
        _mid_body,
        out_shape=(jax.ShapeDtypeStruct((C * N, W), jnp.float32),
                   jax.ShapeDtypeStruct((N,), jnp.float32)),
    )(dro, dci, xh)


def _final_body(u3_ref, nd_ref, bg_ref, w1_ref, b1_ref, w2_ref, b2_ref,
                y_ref):
    bg = bg_ref[...]
    nd = nd_ref[...]

    def channel(c):
        u = u3_ref[pl.ds(c * N, N), :]
        s1 = u[:, 64]
        d1 = jnp.where(s1 == 0, 0.0, 1.0 / jnp.where(s1 == 0, 1.0, s1))
        return jax.nn.relu(u[:, :64] * (nd * d1)[:, None] + bg[None, :])

    x = jnp.concatenate([channel(0), channel(1)], axis=1)
    x = jax.nn.relu(
        jax.lax.dot_general(x, w1_ref[...], (((1,), (0,)), ((), ())),
                            precision=jax.lax.Precision.HIGHEST,
                            preferred_element_type=jnp.float32)
        + b1_ref[...][None, :])
    y_ref[...] = (
        jax.lax.dot_general(x, w2_ref[...], (((1,), (0,)), ((), ())),
                            precision=jax.lax.Precision.HIGHEST,
                            preferred_element_type=jnp.float32)
        + b2_ref[...][None, :])


def _tc_final(u3, nd, bg, w1, b1, w2, b2):
    return pl.pallas_call(
        _final_body,
        out_shape=jax.ShapeDtypeStruct((N, 16), jnp.float32),
    )(u3, nd, bg, w1, b1, w2, b2)


def kernel(h, w_l0_1, w_l0_2, w_l1, W_gcn, b_gcn, W1, b1, W2, b2,
           edge_index, edge_type):
    src = edge_index[0].astype(jnp.int32)
    dst = edge_index[1].astype(jnp.int32)
    typ = edge_type.astype(jnp.int32)
    diag = jnp.arange(N, dtype=jnp.int32)
    src_e = jnp.concatenate([src, diag])
    dst_e = jnp.concatenate([dst, diag])
    typ_e = jnp.concatenate([typ, jnp.full((N,), 4, jnp.int32)])

    ones128 = jnp.ones((CHUNK,), jnp.float32)
    zf = jnp.zeros((16384,), jnp.float32)
    zw = jnp.zeros((SL, W), jnp.float32)

    lut, xh = _tc_prep(w_l0_1, w_l0_2, w_l1, h, W_gcn)
    pflat = _build_p_kernel(src_e, dst_e, ones128, zf)
    pbf = _tc_conv(pflat.reshape(N, N))
    p2bf = _tc_c2(pbf)
    dro, dci = _tc_c3(p2bf, pbf)
    z2, nd = _tc_mid(dro, dci, xh)
    u1, u2, u3 = _value_kernel(src_e, dst_e, typ_e, lut, z2, zw)
    del u1, u2
    return _tc_final(u3, nd, b_gcn, W1, b1, W2, b2)


# speedup vs baseline: 1.0040x; 1.0040x over previous
import functools
import jax
import jax.numpy as jnp
from jax import lax
from jax.experimental import pallas as pl
from jax.experimental.pallas import tpu as pltpu
from jax.experimental.pallas import tpu_sc as plsc

N = 4096
E = 131072
NE = E + N
C = 2
NC, NS, L = 2, 16, 16
EPT = NE // NS
CHUNK = 128
NCHUNK = EPT // CHUNK
NBUF = 3
ROWS_PT = N // (NC * NS)
SL = N // NS
W = 80

_MESH = plsc.VectorSubcoreMesh(core_axis_name="c", subcore_axis_name="s",
                               num_cores=NC, num_subcores=NS)

_SC_PARAMS = pltpu.CompilerParams(needs_layout_passes=False)
_SC_PARAMS_LINEAR = pltpu.CompilerParams(needs_layout_passes=False,
                                         use_tc_tiling_on_sc=False)

_DN = jax.lax.GatherDimensionNumbers(
    offset_dims=(), collapsed_slice_dims=(0,), start_index_map=(0,))


def _vgather16(vec, idx):
    return jax.lax.gather(
        vec, idx[:, None], _DN, slice_sizes=(1,),
        mode=jax.lax.GatherScatterMode.PROMISE_IN_BOUNDS)


def _sc_build_p(src_hbm, dst_hbm, ones_hbm, zf_hbm, p_hbm,
                svb, dvb, iv, onesv):
    cid = lax.axis_index("c")
    sid = lax.axis_index("s")
    base_w = (cid * NS + sid) * ROWS_PT * N

    def zero_body(j, _):
        pltpu.sync_copy(zf_hbm, p_hbm.at[pl.ds(base_w + j * 16384, 16384)])
        return 0
    lax.fori_loop(0, (ROWS_PT * N) // 16384, zero_body, 0)
    plsc.subcore_barrier()

    e0 = sid * EPT
    pltpu.sync_copy(src_hbm.at[pl.ds(e0, EPT)], svb)
    pltpu.sync_copy(dst_hbm.at[pl.ds(e0, EPT)], dvb)
    pltpu.sync_copy(ones_hbm, onesv)

    def chunk_body(i, _):
        for k in range(CHUNK // L):
            sv = svb[pl.ds(i * CHUNK + k * L, L)]
            dv = dvb[pl.ds(i * CHUNK + k * L, L)]
            iv[pl.ds(k * L, L)] = sv * N + dv
        pltpu.sync_copy(onesv, p_hbm.at[iv])
        return 0
    lax.fori_loop(0, NCHUNK, chunk_body, 0)


@functools.partial(
    pl.kernel,
    out_type=jax.ShapeDtypeStruct((N * N,), jnp.float32),
    mesh=_MESH,
    scratch_types=[
        pltpu.VMEM((EPT,), jnp.int32),
        pltpu.VMEM((EPT,), jnp.int32),
        pltpu.VMEM((CHUNK,), jnp.int32),
        pltpu.VMEM((CHUNK,), jnp.float32),
    ],
)
def _build_p_kernel(src_hbm, dst_hbm, ones_hbm, zf_hbm, p_hbm,
                    svb, dvb, iv, onesv):
    _sc_build_p(src_hbm, dst_hbm, ones_hbm, zf_hbm, p_hbm,
                svb, dvb, iv, onesv)


def _sc_value(ed_hbm, lut_hbm, z_hbm, zw_hbm,
              u3_hbm, evb, lutc, slicev, bufs, in_sp, acc_sp):
    cid = lax.axis_index("c")
    sid = lax.axis_index("s")
    iota = jax.lax.iota(jnp.int32, L)
    e0 = sid * EPT
    pltpu.sync_copy(ed_hbm.at[pl.ds(e0, EPT)], evb)
    pltpu.sync_copy(lut_hbm.at[:, cid], lutc)

    row_off = cid * N
    pltpu.sync_copy(zw_hbm, acc_sp.at[pl.ds(sid * SL, SL)])
    pltpu.sync_copy(z_hbm.at[pl.ds(sid * SL, SL)],
                    in_sp.at[pl.ds(sid * SL, SL)])
    plsc.subcore_barrier()
    NV = W // L
    m12 = jnp.full((L,), 0xFFF, jnp.int32)

    def _prep(i, bix, lp):
        idxg, scidx, wbuf, _, _ = bufs[bix]
        for k in range(CHUNK // L):
            ev = evb[pl.ds(i * CHUNK + k * L, L)]
            dv = ev & m12
            sv = jax.lax.shift_right_logical(ev, 12) & m12
            tv = jax.lax.shift_right_logical(ev, 24)
            idxg[pl.ds(k * L, L)] = sv
            scidx[pl.ds(k * L, L)] = dv
            wbuf[pl.ds(k * L, L)] = _vgather16(lp, tv)

    def _start(bix, tbl):
        idxg, _, _, rows, sem = bufs[bix]
        pltpu.async_copy(tbl.at[idxg], rows, sem)

    def _finish(bix, tbl):
        idxg, scidx, wbuf, rows, sem = bufs[bix]
        pltpu.make_async_copy(tbl.at[idxg], rows, sem).wait()

        def q_body(q, _):
            for u in range(4):
                e16 = jnp.full((L,), q * 4 + u, jnp.int32)
                bw = plsc.load_gather(wbuf, [e16])
                for v in range(NV):
                    c16 = iota + (v * L)
                    x = plsc.load_gather(rows, [e16, c16])
                    plsc.store_scatter(rows, [e16, c16], x * bw)
            return 0
        lax.fori_loop(0, CHUNK // 4, q_body, 0)
        pltpu.sync_copy(rows, acc_sp.at[scidx], add=True)

    ones_lane0 = jnp.where(iota == 0, 1.0, 0.0).astype(jnp.float32)

    for p in range(3):
        lp = lutc[p]
        for j in range(NBUF):
            _prep(j, j, lp)
            _start(j, in_sp)

        def ring_body(g, _, lp=lp):
            for j in range(NBUF):
                _finish(j, in_sp)

                @pl.when(g < NCHUNK // NBUF - 1)
                def _(j=j, g=g):
                    _prep((g + 1) * NBUF + j, j, lp)
                    _start(j, in_sp)
            return 0
        lax.fori_loop(0, NCHUNK // NBUF, ring_body, 0)
        plsc.subcore_barrier()

        pltpu.sync_copy(acc_sp.at[pl.ds(sid * SL, SL)], slicev)
        if p == 1:
            def fin_body(r, _):
                r16 = jnp.full((L,), r, jnp.int32)
                s0r = plsc.load_gather(slicev, [r16, jnp.full((L,), 64,
                                                              jnp.int32)])
                inv = jnp.where(s0r == 0, 0.0,
                                1.0 / jnp.where(s0r == 0, 1.0, s0r))
                for v in range(4):
                    c16 = iota + (v * L)
                    x = plsc.load_gather(slicev, [r16, c16])
                    plsc.store_scatter(slicev, [r16, c16], x * inv)
                plsc.store_scatter(slicev, [r16, iota + 4 * L], ones_lane0)
                return 0
            lax.fori_loop(0, SL, fin_body, 0)
        if p == 2:
            pltpu.sync_copy(slicev, u3_hbm.at[pl.ds(row_off + sid * SL, SL)])
        else:
            pltpu.sync_copy(slicev, in_sp.at[pl.ds(sid * SL, SL)])
            pltpu.sync_copy(zw_hbm, acc_sp.at[pl.ds(sid * SL, SL)])
        plsc.subcore_barrier()


def _value_scratch():
    s = [
        pltpu.VMEM((EPT,), jnp.int32),
        pltpu.VMEM((3, L), jnp.float32),
        pltpu.VMEM((SL, W), jnp.float32),
    ]
    for _ in range(NBUF):
        s += [pltpu.VMEM((CHUNK,), jnp.int32),
              pltpu.VMEM((CHUNK,), jnp.int32),
              pltpu.VMEM((CHUNK,), jnp.float32),
              pltpu.VMEM((CHUNK, W), jnp.float32),
              pltpu.SemaphoreType.DMA]
    s.append(pltpu.VMEM_SHARED((N, W), jnp.float32))
    s.append(pltpu.VMEM_SHARED((N, W), jnp.float32))
    return s


@functools.partial(
    pl.kernel,
    out_type=jax.ShapeDtypeStruct((C * N, W), jnp.float32),
    mesh=_MESH,
    compiler_params=_SC_PARAMS_LINEAR,
    scratch_types=_value_scratch(),
)
def _value_kernel(ed_hbm, lut_hbm, z_hbm, zw_hbm,
                  u3_hbm, evb, lutc, slicev, *rest):
    bufs = tuple(tuple(rest[5 * i:5 * i + 5]) for i in range(NBUF))
    in_sp = rest[5 * NBUF]
    acc_sp = rest[5 * NBUF + 1]
    _sc_value(ed_hbm, lut_hbm, z_hbm, zw_hbm,
              u3_hbm, evb, lutc, slicev, bufs, in_sp, acc_sp)


def _prep_body(w1_ref, w2_ref, w3_ref, h_ref, wg_ref, lut_ref, xh_ref):
    def lutify(w):
        f = jax.nn.softmax(w, axis=0)
        return jnp.concatenate([f, jnp.zeros((L - 5, C), f.dtype)], axis=0).T
    lut_ref[0] = lutify(w1_ref[...])
    lut_ref[1] = lutify(w2_ref[...])
    lut_ref[2] = lutify(w3_ref[...])
    xh_ref[...] = jax.lax.dot_general(
        h_ref[...], wg_ref[...], (((1,), (0,)), ((), ())),
        precision=jax.lax.Precision.HIGHEST,
        preferred_element_type=jnp.float32)


def _tc_prep(w1, w2, w3, h, wg):
    return pl.pallas_call(
        _prep_body,
        out_shape=(jax.ShapeDtypeStruct((3, C, L), jnp.float32),
                   jax.ShapeDtypeStruct((N, 64), jnp.float32)),
    )(w1, w2, w3, h, wg)


def _conv_body(p_ref, o_ref):
    o_ref[...] = p_ref[...].astype(jnp.bfloat16)


def _tc_conv(p2d):
    blk = 512
    return pl.pallas_call(
        _conv_body,
        grid=(N // blk,),
        in_specs=[pl.BlockSpec((blk, N), lambda i: (i, 0))],
        out_specs=pl.BlockSpec((blk, N), lambda i: (i, 0)),
        out_shape=jax.ShapeDtypeStruct((N, N), jnp.bfloat16),
    )(p2d)


_BM = 1024


def _c2_body(a_ref, b_ref, o_ref):
    acc = jax.lax.dot_general(
        a_ref[...], b_ref[...], (((1,), (0,)), ((), ())),
        preferred_element_type=jnp.float32)
    o_ref[...] = (acc > 0).astype(jnp.bfloat16)


def _tc_c2(pbf):
    return pl.pallas_call(
        _c2_body,
        grid=(N // _BM, N // _BM),
        in_specs=[pl.BlockSpec((_BM, N), lambda i, j: (i, 0)),
                  pl.BlockSpec((N, _BM), lambda i, j: (0, j))],
        out_specs=pl.BlockSpec((_BM, _BM), lambda i, j: (i, j)),
        out_shape=jax.ShapeDtypeStruct((N, N), jnp.bfloat16),
        compiler_params=pltpu.CompilerParams(
            dimension_semantics=("arbitrary", "arbitrary")),
    )(pbf, pbf)


def _c3_body(a_ref, b_ref, dro_ref, dci_ref, col_acc):
    i = pl.program_id(0)
    j = pl.program_id(1)
    acc = jax.lax.dot_general(
        a_ref[...], b_ref[...], (((1,), (0,)), ((), ())),
        preferred_element_type=jnp.float32)
    ind = (acc > 0).astype(jnp.float32)

    @pl.when(j == 0)
    def _():
        dro_ref[...] = jnp.zeros_like(dro_ref)
    dro_ref[...] += jnp.sum(ind, axis=1)

    @pl.when(jnp.logical_and(i == 0, j == 0))
    def _():
        col_acc[...] = jnp.zeros_like(col_acc)
    col_acc[pl.ds(j * _BM, _BM)] += jnp.sum(ind, axis=0)

    @pl.when(jnp.logical_and(i == pl.num_programs(0) - 1,
                             j == pl.num_programs(1) - 1))
    def _():
        dci_ref[...] = col_acc[...]


def _tc_c3(p2bf, pbf):
    return pl.pallas_call(
        _c3_body,
        grid=(N // _BM, N // _BM),
        in_specs=[pl.BlockSpec((_BM, N), lambda i, j: (i, 0)),
                  pl.BlockSpec((N, _BM), lambda i, j: (0, j))],
        out_specs=(pl.BlockSpec((_BM,), lambda i, j: (i,)),
                   pl.BlockSpec((N,), lambda i, j: (0,))),
        out_shape=(jax.ShapeDtypeStruct((N,), jnp.float32),
                   jax.ShapeDtypeStruct((N,), jnp.float32)),
        scratch_shapes=[pltpu.VMEM((N,), jnp.float32)],
        compiler_params=pltpu.CompilerParams(
            dimension_semantics=("arbitrary", "arbitrary")),
    )(p2bf, pbf)


def _mid_body(dro_ref, dci_ref, xh_ref, z2_ref, nd_ref):
    dro = dro_ref[...]
    dci = dci_ref[...]
    ns = jnp.where(dro > 0, jax.lax.rsqrt(jnp.maximum(dro, 1.0)), 0.0)
    nd_ref[...] = jnp.where(dci > 0, jax.lax.rsqrt(jnp.maximum(dci, 1.0)),
                            0.0)
    z = ns[:, None] * xh_ref[...]
    col = jnp.concatenate(
        [jnp.ones((N, 1), jnp.float32), jnp.zeros((N, W - 65), jnp.float32)],
        axis=1)
    z2_ref[...] = jnp.concatenate([z, col], axis=1)


def _tc_mid(dro, dci, xh):
    return pl.pallas_call(
        _mid_body,
        out_shape=(jax.ShapeDtypeStruct((N, W), jnp.float32),
                   jax.ShapeDtypeStruct((N,), jnp.float32)),
    )(dro, dci, xh)


def _final_body(u3_ref, nd_ref, bg_ref, w1_ref, b1_ref, w2_ref, b2_ref,
                y_ref):
    bg = bg_ref[...]
    nd = nd_ref[...]

    def channel(c):
        u = u3_ref[pl.ds(c * N, N), :]
        s1 = u[:, 64]
        d1 = jnp.where(s1 == 0, 0.0, 1.0 / jnp.where(s1 == 0, 1.0, s1))
        return jax.nn.relu(u[:, :64] * (nd * d1)[:, None] + bg[None, :])

    x = jnp.concatenate([channel(0), channel(1)], axis=1)
    x = jax.nn.relu(
        jax.lax.dot_general(x, w1_ref[...], (((1,), (0,)), ((), ())),
                            precision=jax.lax.Precision.HIGHEST,
                            preferred_element_type=jnp.float32)
        + b1_ref[...][None, :])
    y_ref[...] = (
        jax.lax.dot_general(x, w2_ref[...], (((1,), (0,)), ((), ())),
                            precision=jax.lax.Precision.HIGHEST,
                            preferred_element_type=jnp.float32)
        + b2_ref[...][None, :])


def _tc_final(u3, nd, bg, w1, b1, w2, b2):
    return pl.pallas_call(
        _final_body,
        out_shape=jax.ShapeDtypeStruct((N, 16), jnp.float32),
    )(u3, nd, bg, w1, b1, w2, b2)


def kernel(h, w_l0_1, w_l0_2, w_l1, W_gcn, b_gcn, W1, b1, W2, b2,
           edge_index, edge_type):
    src = edge_index[0].astype(jnp.int32)
    dst = edge_index[1].astype(jnp.int32)
    typ = edge_type.astype(jnp.int32)
    diag = jnp.arange(N, dtype=jnp.int32)
    src_e = jnp.concatenate([src, diag])
    dst_e = jnp.concatenate([dst, diag])
    typ_e = jnp.concatenate([typ, jnp.full((N,), 4, jnp.int32)])

    ones128 = jnp.ones((CHUNK,), jnp.float32)
    zf = jnp.zeros((16384,), jnp.float32)
    zw = jnp.zeros((SL, W), jnp.float32)

    ed = (typ_e << 24) | (src_e << 12) | dst_e

    lut, xh = _tc_prep(w_l0_1, w_l0_2, w_l1, h, W_gcn)
    pflat = _build_p_kernel(src_e, dst_e, ones128, zf)
    pbf = _tc_conv(pflat.reshape(N, N))
    p2bf = _tc_c2(pbf)
    dro, dci = _tc_c3(p2bf, pbf)
    z2, nd = _tc_mid(dro, dci, xh)
    ed = pltpu.with_memory_space_constraint(ed, pltpu.MemorySpace.HBM)
    z2 = pltpu.with_memory_space_constraint(z2, pltpu.MemorySpace.HBM)
    zw = pltpu.with_memory_space_constraint(zw, pltpu.MemorySpace.HBM)
    u3 = _value_kernel(ed, lut, z2, zw)
    return _tc_final(u3, nd, b_gcn, W1, b1, W2, b2)
